# Initial kernel scaffold; baseline (speedup 1.0000x reference)
import jax
import jax.numpy as jnp
from jax.experimental import pallas as pl


def kernel(images, features, w_conv, b_conv, w_obj, b_obj, w_box, b_box):
    raise NotImplementedError("write your pallas kernel here")


# speedup vs baseline: 22.0945x; 22.0945x over previous
import functools

import jax
import jax.numpy as jnp
import numpy as np
from jax.experimental import pallas as pl

H = 48
W = 48
STRIDE = 16
A = 9
C_IN = 256
N_ANCH = H * W * A
ROWS = N_ANCH // 128
PRE_NMS = 6000
POST_NMS = 1000
NMS_THRESH = 0.7
MIN_SIZE = 1e-3
BBOX_CLIP = float(np.log(1000.0 / 16.0))
NEG_INF = float("-inf")


def _conv_heads_kernel(x_ref, w9_ref, bconv_ref, wh_ref, bh_ref, out_ref):
    acc = jnp.zeros((H * W, C_IN), dtype=jnp.float32)
    for j in range(9):
        dh, dw = j // 3, j % 3
        xs = x_ref[dh:dh + H, dw:dw + W, :].reshape(H * W, C_IN)
        acc = acc + jnp.dot(xs, w9_ref[j], preferred_element_type=jnp.float32)
    t = jnp.maximum(acc + bconv_ref[...], 0.0)
    out_ref[...] = jnp.dot(t, wh_ref[...],
                           preferred_element_type=jnp.float32) + bh_ref[...]


def _nms_kernel(s_ref, dx_ref, dy_ref, dw_ref, dh_ref,
                aw_ref, ah_ref, acx_ref, acy_ref,
                ox1_ref, oy1_ref, ox2_ref, oy2_ref, osc_ref,
                *, img_w, img_h):
    S = s_ref[...]
    aw = aw_ref[...]
    ah = ah_ref[...]
    dwc = jnp.minimum(dw_ref[...], BBOX_CLIP)
    dhc = jnp.minimum(dh_ref[...], BBOX_CLIP)
    pcx = dx_ref[...] * aw + acx_ref[...]
    pcy = dy_ref[...] * ah + acy_ref[...]
    pw = jnp.exp(dwc) * aw
    ph = jnp.exp(dhc) * ah
    X1 = jnp.clip(pcx - 0.5 * pw, 0.0, img_w)
    Y1 = jnp.clip(pcy - 0.5 * ph, 0.0, img_h)
    X2 = jnp.clip(pcx + 0.5 * pw, 0.0, img_w)
    Y2 = jnp.clip(pcy + 0.5 * ph, 0.0, img_h)
    bw = X2 - X1
    bh = Y2 - Y1
    valid = jnp.logical_and(bw >= MIN_SIZE, bh >= MIN_SIZE)
    areas = bw * bh

    b = jax.lax.bitcast_convert_type(S, jnp.int32)
    key = b ^ ((b >> 31) & jnp.int32(0x7FFFFFFF))
    kmin = jnp.min(key) - 1
    kmax = jnp.max(key)

    def _bs_val(_, st):
        lo, hi = st
        mid = (lo & hi) + ((lo ^ hi) >> 1)
        g = jnp.sum(jnp.where(key > mid, 1, 0).astype(jnp.int32))
        take_lo = g >= PRE_NMS
        return (jnp.where(take_lo, mid, lo), jnp.where(take_lo, hi, mid))

    _, thr = jax.lax.fori_loop(0, 33, _bs_val, (kmin, kmax))
    g_cnt = jnp.sum(jnp.where(key > thr, 1, 0).astype(jnp.int32))
    eq = key == thr

    rows_i = jax.lax.broadcasted_iota(jnp.int32, (ROWS, 128), 0)
    cols_i = jax.lax.broadcasted_iota(jnp.int32, (ROWS, 128), 1)
    iota = rows_i * 128 + cols_i

    def _bs_idx(_, st):
        lo, hi = st
        mid = (lo + hi) // 2
        c = g_cnt + jnp.sum(
            jnp.where(jnp.logical_and(eq, iota <= mid), 1, 0).astype(jnp.int32))
        ok = c >= PRE_NMS
        return (jnp.where(ok, lo, mid), jnp.where(ok, mid, hi))

    _, cutoff = jax.lax.fori_loop(0, 16, _bs_idx,
                                  (jnp.int32(-1), jnp.int32(N_ANCH - 1)))
    sel = jnp.logical_or(key > thr, jnp.logical_and(eq, iota <= cutoff))

    neg = jnp.float32(NEG_INF)
    s0 = jnp.where(jnp.logical_and(sel, valid),
                   jax.nn.sigmoid(S), neg)

    big = jnp.int32(1 << 30)

    def _nms_body(i, s):
        m = jnp.max(s)
        validm = m > neg
        idx = jnp.min(jnp.where(s == m, iota, big))
        onehot = iota == idx
        bx1 = jnp.sum(jnp.where(onehot, X1, 0.0))
        by1 = jnp.sum(jnp.where(onehot, Y1, 0.0))
        bx2 = jnp.sum(jnp.where(onehot, X2, 0.0))
        by2 = jnp.sum(jnp.where(onehot, Y2, 0.0))
        a1 = (bx2 - bx1) * (by2 - by1)
        iw = jnp.maximum(jnp.minimum(bx2, X2) - jnp.maximum(bx1, X1), 0.0)
        ih = jnp.maximum(jnp.minimum(by2, Y2) - jnp.maximum(by1, Y1), 0.0)
        inter = iw * ih
        suppress = inter > NMS_THRESH * (a1 + areas - inter + 1e-9)
        s2 = jnp.where(
            jnp.logical_or(jnp.logical_and(suppress, validm), onehot), neg, s)
        zrow = jnp.zeros((1, 128), dtype=jnp.float32)
        fv = jnp.where(validm, 1.0, 0.0)
        ox1_ref[pl.ds(i, 1), :] = zrow + bx1 * fv
        oy1_ref[pl.ds(i, 1), :] = zrow + by1 * fv
        ox2_ref[pl.ds(i, 1), :] = zrow + bx2 * fv
        oy2_ref[pl.ds(i, 1), :] = zrow + by2 * fv
        osc_ref[pl.ds(i, 1), :] = zrow + jnp.where(validm, m, 0.0)
        return s2

    jax.lax.fori_loop(0, POST_NMS, _nms_body, s0)


def _anchor_planes():
    sizes = np.array([32.0, 64.0, 128.0], dtype=np.float32)
    ratios = np.array([0.5, 1.0, 2.0], dtype=np.float32)
    h_r = np.sqrt(ratios)
    w_r = 1.0 / h_r
    ws = (w_r[:, None] * sizes[None, :]).reshape(-1)
    hs = (h_r[:, None] * sizes[None, :]).reshape(-1)
    cell = np.round(np.stack([-ws, -hs, ws, hs], axis=1) / 2.0).astype(np.float32)
    sx = np.arange(W, dtype=np.float32) * STRIDE
    sy = np.arange(H, dtype=np.float32) * STRIDE
    gy, gx = np.meshgrid(sy, sx, indexing="ij")
    shifts = np.stack([gx.reshape(-1), gy.reshape(-1),
                       gx.reshape(-1), gy.reshape(-1)], axis=1)
    anch = (shifts[:, None, :] + cell[None, :, :]).reshape(-1, 4)
    widths = anch[:, 2] - anch[:, 0]
    heights = anch[:, 3] - anch[:, 1]
    ctr_x = anch[:, 0] + 0.5 * widths
    ctr_y = anch[:, 1] + 0.5 * heights
    shape = (ROWS, 128)
    return (jnp.asarray(widths.reshape(shape)),
            jnp.asarray(heights.reshape(shape)),
            jnp.asarray(ctr_x.reshape(shape)),
            jnp.asarray(ctr_y.reshape(shape)))


def kernel(images, features, w_conv, b_conv, w_obj, b_obj, w_box, b_box):
    img_h = float(images.shape[2])
    img_w = float(images.shape[3])

    x = jnp.transpose(features[0], (1, 2, 0))
    xpad = jnp.pad(x, ((1, 1), (1, 1), (0, 0)))
    w9 = jnp.transpose(w_conv, (2, 3, 1, 0)).reshape(9, C_IN, C_IN)
    wobj = jnp.transpose(w_obj[:, :, 0, 0], (1, 0))
    wbox = jnp.transpose(w_box[:, :, 0, 0], (1, 0))
    wh = jnp.concatenate(
        [wobj, wbox, jnp.zeros((C_IN, 128 - A - 4 * A), jnp.float32)], axis=1)
    bh = jnp.concatenate(
        [b_obj, b_box, jnp.zeros((128 - A - 4 * A,), jnp.float32)])[None, :]

    heads = pl.pallas_call(
        _conv_heads_kernel,
        out_shape=jax.ShapeDtypeStruct((H * W, 128), jnp.float32),
    )(xpad, w9, b_conv[None, :], wh, bh)

    obj = heads[:, :A].reshape(ROWS, 128)
    deltas = heads[:, A:A + 4 * A].reshape(H * W, A, 4)
    dxp = deltas[:, :, 0].reshape(ROWS, 128)
    dyp = deltas[:, :, 1].reshape(ROWS, 128)
    dwp = deltas[:, :, 2].reshape(ROWS, 128)
    dhp = deltas[:, :, 3].reshape(ROWS, 128)
    aw, ah, acx, acy = _anchor_planes()

    out_shapes = [jax.ShapeDtypeStruct((POST_NMS, 128), jnp.float32)] * 5
    ox1, oy1, ox2, oy2, osc = pl.pallas_call(
        functools.partial(_nms_kernel, img_w=img_w, img_h=img_h),
        out_shape=out_shapes,
    )(obj, dxp, dyp, dwp, dhp, aw, ah, acx, acy)

    out_boxes = jnp.stack(
        [ox1[:, 0], oy1[:, 0], ox2[:, 0], oy2[:, 0]], axis=1)
    out_scores = osc[:, 0]
    return out_boxes, out_scores


# speedup vs baseline: 23.7486x; 1.0749x over previous
import functools

import jax
import jax.numpy as jnp
import numpy as np
from jax.experimental import pallas as pl
from jax.experimental.pallas import tpu as pltpu

H = 48
W = 48
STRIDE = 16
A = 9
C_IN = 256
N_ANCH = H * W * A
ROWS = N_ANCH // 128
PRE_NMS = 6000
POST_NMS = 1000
NMS_THRESH = 0.7
MIN_SIZE = 1e-3
BBOX_CLIP = float(np.log(1000.0 / 16.0))
NEG_INF = float("-inf")


def _conv_heads_kernel(x_ref, w9_ref, bconv_ref, wh_ref, bh_ref, out_ref):
    acc = jnp.zeros((H * W, C_IN), dtype=jnp.float32)
    for j in range(9):
        dh, dw = j // 3, j % 3
        xs = x_ref[dh:dh + H, dw:dw + W, :].reshape(H * W, C_IN)
        acc = acc + jnp.dot(xs, w9_ref[j], preferred_element_type=jnp.float32)
    t = jnp.maximum(acc + bconv_ref[...], 0.0)
    out_ref[...] = jnp.dot(t, wh_ref[...],
                           preferred_element_type=jnp.float32) + bh_ref[...]


def _nms_kernel(s_ref, dx_ref, dy_ref, dw_ref, dh_ref,
                aw_ref, ah_ref, acx_ref, acy_ref,
                ox1_ref, oy1_ref, ox2_ref, oy2_ref, osc_ref,
                sx1_ref, sy1_ref, sx2_ref, sy2_ref,
                *, img_w, img_h):
    S = s_ref[...]
    aw = aw_ref[...]
    ah = ah_ref[...]
    dwc = jnp.minimum(dw_ref[...], BBOX_CLIP)
    dhc = jnp.minimum(dh_ref[...], BBOX_CLIP)
    pcx = dx_ref[...] * aw + acx_ref[...]
    pcy = dy_ref[...] * ah + acy_ref[...]
    pw = jnp.exp(dwc) * aw
    ph = jnp.exp(dhc) * ah
    X1 = jnp.clip(pcx - 0.5 * pw, 0.0, img_w)
    Y1 = jnp.clip(pcy - 0.5 * ph, 0.0, img_h)
    X2 = jnp.clip(pcx + 0.5 * pw, 0.0, img_w)
    Y2 = jnp.clip(pcy + 0.5 * ph, 0.0, img_h)
    bw = X2 - X1
    bh = Y2 - Y1
    valid = jnp.logical_and(bw >= MIN_SIZE, bh >= MIN_SIZE)
    areas = bw * bh

    b = jax.lax.bitcast_convert_type(S, jnp.int32)
    key = b ^ ((b >> 31) & jnp.int32(0x7FFFFFFF))
    kmin = jnp.min(key) - 1
    kmax = jnp.max(key)

    def _bs_val(_, st):
        lo, hi = st
        mid = (lo & hi) + ((lo ^ hi) >> 1)
        g = jnp.sum(jnp.where(key > mid, 1, 0).astype(jnp.int32))
        take_lo = g >= PRE_NMS
        return (jnp.where(take_lo, mid, lo), jnp.where(take_lo, hi, mid))

    _, thr = jax.lax.fori_loop(0, 33, _bs_val, (kmin, kmax))
    g_cnt = jnp.sum(jnp.where(key > thr, 1, 0).astype(jnp.int32))
    eq = key == thr

    rows_i = jax.lax.broadcasted_iota(jnp.int32, (ROWS, 128), 0)
    cols_i = jax.lax.broadcasted_iota(jnp.int32, (ROWS, 128), 1)
    iota = rows_i * 128 + cols_i

    def _bs_idx(_, st):
        lo, hi = st
        mid = (lo + hi) // 2
        c = g_cnt + jnp.sum(
            jnp.where(jnp.logical_and(eq, iota <= mid), 1, 0).astype(jnp.int32))
        ok = c >= PRE_NMS
        return (jnp.where(ok, lo, mid), jnp.where(ok, mid, hi))

    _, cutoff = jax.lax.fori_loop(0, 16, _bs_idx,
                                  (jnp.int32(-1), jnp.int32(N_ANCH - 1)))
    sel = jnp.logical_or(key > thr, jnp.logical_and(eq, iota <= cutoff))

    neg = jnp.float32(NEG_INF)
    s0 = jnp.where(jnp.logical_and(sel, valid),
                   jax.nn.sigmoid(S), neg)

    sx1_ref[...] = X1
    sy1_ref[...] = Y1
    sx2_ref[...] = X2
    sy2_ref[...] = Y2
    lane = jax.lax.broadcasted_iota(jnp.int32, (1, 128), 1)
    big = jnp.int32(1 << 30)

    def _nms_body(i, s):
        m = jnp.max(s)
        validm = m > neg
        idx = jnp.min(jnp.where(s == m, iota, big))
        r = idx // 128
        oh = (lane == (idx - r * 128)).astype(jnp.float32)
        bx1 = jnp.sum(sx1_ref[pl.ds(r, 1), :] * oh)
        by1 = jnp.sum(sy1_ref[pl.ds(r, 1), :] * oh)
        bx2 = jnp.sum(sx2_ref[pl.ds(r, 1), :] * oh)
        by2 = jnp.sum(sy2_ref[pl.ds(r, 1), :] * oh)
        a1 = (bx2 - bx1) * (by2 - by1)
        iw = jnp.maximum(jnp.minimum(bx2, X2) - jnp.maximum(bx1, X1), 0.0)
        ih = jnp.maximum(jnp.minimum(by2, Y2) - jnp.maximum(by1, Y1), 0.0)
        inter = iw * ih
        s2 = jnp.where(inter > NMS_THRESH * (a1 + areas - inter + 1e-9), neg, s)
        zrow = jnp.zeros((1, 128), dtype=jnp.float32)
        fv = jnp.where(validm, 1.0, 0.0)
        ox1_ref[pl.ds(i, 1), :] = zrow + bx1 * fv
        oy1_ref[pl.ds(i, 1), :] = zrow + by1 * fv
        ox2_ref[pl.ds(i, 1), :] = zrow + bx2 * fv
        oy2_ref[pl.ds(i, 1), :] = zrow + by2 * fv
        osc_ref[pl.ds(i, 1), :] = zrow + jnp.where(validm, m, 0.0)
        return s2

    jax.lax.fori_loop(0, POST_NMS, _nms_body, s0)


def _anchor_planes():
    sizes = np.array([32.0, 64.0, 128.0], dtype=np.float32)
    ratios = np.array([0.5, 1.0, 2.0], dtype=np.float32)
    h_r = np.sqrt(ratios)
    w_r = 1.0 / h_r
    ws = (w_r[:, None] * sizes[None, :]).reshape(-1)
    hs = (h_r[:, None] * sizes[None, :]).reshape(-1)
    cell = np.round(np.stack([-ws, -hs, ws, hs], axis=1) / 2.0).astype(np.float32)
    sx = np.arange(W, dtype=np.float32) * STRIDE
    sy = np.arange(H, dtype=np.float32) * STRIDE
    gy, gx = np.meshgrid(sy, sx, indexing="ij")
    shifts = np.stack([gx.reshape(-1), gy.reshape(-1),
                       gx.reshape(-1), gy.reshape(-1)], axis=1)
    anch = (shifts[:, None, :] + cell[None, :, :]).reshape(-1, 4)
    widths = anch[:, 2] - anch[:, 0]
    heights = anch[:, 3] - anch[:, 1]
    ctr_x = anch[:, 0] + 0.5 * widths
    ctr_y = anch[:, 1] + 0.5 * heights
    shape = (ROWS, 128)
    return (jnp.asarray(widths.reshape(shape)),
            jnp.asarray(heights.reshape(shape)),
            jnp.asarray(ctr_x.reshape(shape)),
            jnp.asarray(ctr_y.reshape(shape)))


def kernel(images, features, w_conv, b_conv, w_obj, b_obj, w_box, b_box):
    img_h = float(images.shape[2])
    img_w = float(images.shape[3])

    x = jnp.transpose(features[0], (1, 2, 0))
    xpad = jnp.pad(x, ((1, 1), (1, 1), (0, 0)))
    w9 = jnp.transpose(w_conv, (2, 3, 1, 0)).reshape(9, C_IN, C_IN)
    wobj = jnp.transpose(w_obj[:, :, 0, 0], (1, 0))
    wbox = jnp.transpose(w_box[:, :, 0, 0], (1, 0))
    wh = jnp.concatenate(
        [wobj, wbox, jnp.zeros((C_IN, 128 - A - 4 * A), jnp.float32)], axis=1)
    bh = jnp.concatenate(
        [b_obj, b_box, jnp.zeros((128 - A - 4 * A,), jnp.float32)])[None, :]

    heads = pl.pallas_call(
        _conv_heads_kernel,
        out_shape=jax.ShapeDtypeStruct((H * W, 128), jnp.float32),
    )(xpad, w9, b_conv[None, :], wh, bh)

    obj = heads[:, :A].reshape(ROWS, 128)
    deltas = heads[:, A:A + 4 * A].reshape(H * W, A, 4)
    dxp = deltas[:, :, 0].reshape(ROWS, 128)
    dyp = deltas[:, :, 1].reshape(ROWS, 128)
    dwp = deltas[:, :, 2].reshape(ROWS, 128)
    dhp = deltas[:, :, 3].reshape(ROWS, 128)
    aw, ah, acx, acy = _anchor_planes()

    out_shapes = [jax.ShapeDtypeStruct((POST_NMS, 128), jnp.float32)] * 5
    ox1, oy1, ox2, oy2, osc = pl.pallas_call(
        functools.partial(_nms_kernel, img_w=img_w, img_h=img_h),
        out_shape=out_shapes,
        scratch_shapes=[pltpu.VMEM((ROWS, 128), jnp.float32)] * 4,
    )(obj, dxp, dyp, dwp, dhp, aw, ah, acx, acy)

    out_boxes = jnp.stack(
        [ox1[:, 0], oy1[:, 0], ox2[:, 0], oy2[:, 0]], axis=1)
    out_scores = osc[:, 0]
    return out_boxes, out_scores
